# baseline (speedup 1.0000x reference)
import functools

import jax
import jax.numpy as jnp
from jax import lax
from jax.experimental import pallas as pl
from jax.experimental.pallas import tpu as pltpu
from jax.experimental.pallas import tpu_sc as plsc

_f32 = jnp.float32
_i32 = jnp.int32

N = 50000
E = 800000
VOC = 10000
D = 64
DH = 32
G = 256
NCLS = 2

NC = 2
NS = 16
NW = NC * NS

NPAD = 50176
PT_N = NPAD // NW
EPAD = 811008
PT_E = EPAD // NS
PT_E0 = EPAD // NW
CAP = 264
RPT = NPAD // NS
NCH = PT_E // 128
GK = 3
SLAB = 18
NSLAB = NCH // SLAB
NCH0 = PT_E0 // 128

_mesh = plsc.VectorSubcoreMesh(core_axis_name="c", subcore_axis_name="s")
_sc_params = pltpu.CompilerParams(use_tc_tiling_on_sc=False)


def _fill_vec(ref, n16, value):
    def body(i, carry):
        ref[pl.ds(i * 16, 16)] = jnp.full((16,), value, _f32)
        return carry
    lax.fori_loop(0, n16, body, 0)


def _fill_rows(ref, nrows, ncol, value):
    def body(i, carry):
        for k in range(ncol // 16):
            ref[i, pl.ds(k * 16, 16)] = jnp.full((16,), value, _f32)
        return carry
    lax.fori_loop(0, nrows, body, 0)


@functools.partial(
    pl.kernel,
    out_type=(jax.ShapeDtypeStruct((2 * NPAD,), _f32),
              jax.ShapeDtypeStruct((2 * CAP,), _f32),
              jax.ShapeDtypeStruct((NPAD, D), _f32)),
    mesh=_mesh,
    compiler_params=_sc_params,
    scratch_types=(
        pltpu.VMEM((NCH0, 128), _i32),
        pltpu.VMEM((PT_N // 112, 112), _i32),
        pltpu.VMEM((128,), _f32),
        pltpu.VMEM((112,), _f32),
        pltpu.VMEM((112,), _f32),
        pltpu.VMEM((112,), _i32),
        pltpu.VMEM((112, D), _f32),
        pltpu.VMEM_SHARED((NPAD,), _f32),
        pltpu.VMEM_SHARED((CAP,), _f32),
        pltpu.SemaphoreType.DMA,
    ),
)
def _prep(dst_hbm, batch_hbm, t1_hbm, x_hbm, dego_hbm, cnto_hbm, h1_hbm,
          dstall, ball, ones_v, zero_v, bounce, xidx, rows, dacc, cacc, sem):
    c = lax.axis_index("c")
    s = lax.axis_index("s")
    wid = s * NC + c
    _fill_vec(ones_v, 8, 1.0)
    _fill_vec(zero_v, 7, 0.0)
    pltpu.sync_copy(dst_hbm.at[c, s], dstall)
    pltpu.sync_copy(batch_hbm.at[c, s], ball)

    def zdeg(i, carry):
        pltpu.sync_copy(zero_v, dacc.at[pl.ds(s * RPT + i * 112, 112)])
        return carry
    lax.fori_loop(0, RPT // 112, zdeg, 0)

    @pl.when(s == 0)
    def _():
        pltpu.sync_copy(zero_v, cacc.at[pl.ds(0, 112)])
        pltpu.sync_copy(zero_v, cacc.at[pl.ds(112, 112)])
        pltpu.sync_copy(zero_v.at[pl.ds(0, 40)], cacc.at[pl.ds(224, 40)])

    plsc.subcore_barrier()

    def edges(j, carry):
        pltpu.sync_copy(ones_v, dacc.at[dstall.at[j]], add=True)
        return carry
    lax.fori_loop(0, NCH0, edges, 0)

    def nodes(j, carry):
        pltpu.sync_copy(ones_v.at[pl.ds(0, 112)], cacc.at[ball.at[j]], add=True)
        return carry
    lax.fori_loop(0, PT_N // 112, nodes, 0)

    def emb(j, carry):
        base = wid * PT_N + j * 112
        pltpu.sync_copy(x_hbm.at[pl.ds(base, 112)], xidx)
        pltpu.async_copy(t1_hbm.at[xidx], rows, sem).wait()
        pltpu.sync_copy(rows, h1_hbm.at[pl.ds(base, 112)])
        return carry
    lax.fori_loop(0, PT_N // 112, emb, 0)

    plsc.subcore_barrier()

    def wdeg(i, carry):
        r0 = s * RPT + i * 112
        pltpu.sync_copy(dacc.at[pl.ds(r0, 112)], bounce)
        pltpu.sync_copy(bounce, dego_hbm.at[pl.ds(c * NPAD + r0, 112)])
        return carry
    lax.fori_loop(0, RPT // 112, wdeg, 0)

    @pl.when(s == 0)
    def _():
        pltpu.sync_copy(cacc.at[pl.ds(0, 112)], bounce)
        pltpu.sync_copy(bounce, cnto_hbm.at[pl.ds(c * CAP, 112)])
        pltpu.sync_copy(cacc.at[pl.ds(112, 112)], bounce)
        pltpu.sync_copy(bounce, cnto_hbm.at[pl.ds(c * CAP + 112, 112)])
        pltpu.sync_copy(cacc.at[pl.ds(224, 40)], bounce.at[pl.ds(0, 40)])
        pltpu.sync_copy(bounce.at[pl.ds(0, 40)],
                        cnto_hbm.at[pl.ds(c * CAP + 224, 40)])


@functools.partial(
    pl.kernel,
    out_type=jax.ShapeDtypeStruct((2 * NPAD, DH), _f32),
    mesh=_mesh,
    compiler_params=_sc_params,
    scratch_types=(
        pltpu.VMEM((SLAB, 128), _i32),
        pltpu.VMEM((SLAB, 128), _i32),
        pltpu.VMEM((2 * GK, 128, DH), _f32),
        pltpu.VMEM_SHARED((NPAD, DH), _f32),
        pltpu.SemaphoreType.DMA,
        pltpu.SemaphoreType.DMA,
    ),
)
def _agg(g_hbm, src_hbm, dst_hbm, out_hbm,
         sidx, didx, rows, acc, sem, ssem):
    c = lax.axis_index("c")
    s = lax.axis_index("s")
    _fill_rows(rows.at[0], 128, DH, 0.0)
    zrow = rows.at[0].at[pl.ds(0, 112)]

    def zacc(i, carry):
        pltpu.sync_copy(zrow, acc.at[pl.ds(s * RPT + i * 112, 112)])
        return carry
    lax.fori_loop(0, RPT // 112, zacc, 0)

    plsc.subcore_barrier()

    def fire(grp, par):
        for b in range(GK):
            pltpu.async_copy(g_hbm.at[c].at[sidx.at[grp * GK + b]],
                             rows.at[par * GK + b], sem)

    NGRP = SLAB // GK

    def drain_scatter(par):
        for b in range(GK):
            pltpu.make_async_copy(rows.at[par * GK + b],
                                  acc.at[pl.ds(0, 128)], ssem).wait()

    for sl in range(NSLAB):
        pltpu.sync_copy(src_hbm.at[s, sl], sidx)
        pltpu.sync_copy(dst_hbm.at[s, sl], didx)
        fire(0, 0)

        def dbl(t, carry):
            for par in (0, 1):
                grp = t * 2 + par
                for b in range(GK):
                    pltpu.make_async_copy(g_hbm.at[0].at[pl.ds(0, 128)],
                                          rows.at[par * GK + b], sem).wait()

                @pl.when(grp >= 1)
                def _():
                    drain_scatter(1 - par)

                @pl.when(grp + 1 < NGRP)
                def _():
                    fire(grp + 1, 1 - par)

                for b in range(GK):
                    pltpu.async_copy(rows.at[par * GK + b],
                                     acc.at[didx.at[grp * GK + b]], ssem,
                                     add=True)
            return carry
        lax.fori_loop(0, NGRP // 2, dbl, 0)
        drain_scatter(1)

    plsc.subcore_barrier()

    def wout(i, carry):
        r0 = s * RPT + i * 112
        zb = rows.at[0].at[pl.ds(0, 112)]
        pltpu.sync_copy(acc.at[pl.ds(r0, 112)], zb)
        pltpu.sync_copy(zb, out_hbm.at[pl.ds(c * NPAD + r0, 112)])
        return carry
    lax.fori_loop(0, RPT // 112, wout, 0)


@functools.partial(
    pl.kernel,
    out_type=jax.ShapeDtypeStruct((2 * CAP, D), _f32),
    mesh=_mesh,
    compiler_params=_sc_params,
    scratch_types=(
        pltpu.VMEM((112,), _i32),
        pltpu.VMEM((112, D), _f32),
        pltpu.VMEM((112, D), _f32),
        pltpu.VMEM_SHARED((CAP, D), _f32),
    ),
)
def _pool(h2_hbm, batch_hbm, out_hbm, bidx, rows, zrow, pacc):
    c = lax.axis_index("c")
    s = lax.axis_index("s")
    _fill_rows(zrow, 112, D, 0.0)

    @pl.when(s == 0)
    def _():
        pltpu.sync_copy(zrow, pacc.at[pl.ds(0, 112)])
        pltpu.sync_copy(zrow, pacc.at[pl.ds(112, 112)])
        pltpu.sync_copy(zrow.at[pl.ds(0, 40)], pacc.at[pl.ds(224, 40)])

    plsc.subcore_barrier()

    def body(j, carry):
        base = c * (NPAD // 2) + s * PT_N + j * 112
        pltpu.sync_copy(batch_hbm.at[pl.ds(base, 112)], bidx)
        pltpu.sync_copy(h2_hbm.at[pl.ds(base, 112)], rows)
        pltpu.sync_copy(rows, pacc.at[bidx], add=True)
        return carry
    lax.fori_loop(0, PT_N // 112, body, 0)

    plsc.subcore_barrier()

    @pl.when(s == 0)
    def _():
        pltpu.sync_copy(pacc.at[pl.ds(0, 112)], zrow)
        pltpu.sync_copy(zrow, out_hbm.at[pl.ds(c * CAP, 112)])
        pltpu.sync_copy(pacc.at[pl.ds(112, 112)], zrow)
        pltpu.sync_copy(zrow, out_hbm.at[pl.ds(c * CAP + 112, 112)])
        pltpu.sync_copy(pacc.at[pl.ds(224, 40)], zrow.at[pl.ds(0, 40)])
        pltpu.sync_copy(zrow.at[pl.ds(0, 40)],
                        out_hbm.at[pl.ds(c * CAP + 224, 40)])


RB = NPAD // 14


def _t1_body(emb_ref, w_ref, out_ref):
    out_ref[...] = lax.dot_general(emb_ref[...], w_ref[...],
                                   (((1,), (1,)), ((), ())),
                                   preferred_element_type=_f32)


def _t1_call(embed, W1):
    return pl.pallas_call(
        _t1_body,
        out_shape=jax.ShapeDtypeStruct((VOC, D), _f32),
    )(embed, W1)


def _t2_body(h_ref, d_ref, out_ref):
    dis = lax.rsqrt(d_ref[0] + d_ref[1] + 1.0)
    g = h_ref[...] * dis[:, None]
    out_ref[0] = g[:, :DH]
    out_ref[1] = g[:, DH:]


def _t2_call(hhat1, dpart):
    return pl.pallas_call(
        _t2_body,
        grid=(14,),
        in_specs=[
            pl.BlockSpec((RB, D), lambda i: (i, 0)),
            pl.BlockSpec((2, RB), lambda i: (0, i)),
        ],
        out_specs=pl.BlockSpec((2, RB, DH), lambda i: (0, i, 0)),
        out_shape=jax.ShapeDtypeStruct((2, NPAD, DH), _f32),
    )(hhat1, dpart)


def _t4_body(a_ref, g_ref, d_ref, w_ref, b_ref, out_ref):
    dis = lax.rsqrt(d_ref[0] + d_ref[1] + 1.0)
    agg = jnp.concatenate([a_ref[0], a_ref[1]], axis=1)
    g1 = jnp.concatenate([g_ref[0], g_ref[1]], axis=1)
    h1 = jnp.maximum((agg + g1) * dis[:, None] + b_ref[...], 0.0)
    hh2 = lax.dot_general(h1, w_ref[...], (((1,), (1,)), ((), ())),
                          preferred_element_type=_f32)
    g2 = hh2 * dis[:, None]
    out_ref[0] = g2[:, :DH]
    out_ref[1] = g2[:, DH:]


def _t4_call(agg1, g1, dpart, W2, b1):
    return pl.pallas_call(
        _t4_body,
        grid=(14,),
        in_specs=[
            pl.BlockSpec((2, RB, DH), lambda i: (0, i, 0)),
            pl.BlockSpec((2, RB, DH), lambda i: (0, i, 0)),
            pl.BlockSpec((2, RB), lambda i: (0, i)),
            pl.BlockSpec((D, D), lambda i: (0, 0)),
            pl.BlockSpec((1, D), lambda i: (0, 0)),
        ],
        out_specs=pl.BlockSpec((2, RB, DH), lambda i: (0, i, 0)),
        out_shape=jax.ShapeDtypeStruct((2, NPAD, DH), _f32),
    )(agg1, g1, dpart, W2, b1)


def _t5_body(a_ref, g_ref, d_ref, b_ref, out_ref):
    dis = lax.rsqrt(d_ref[0] + d_ref[1] + 1.0)
    agg = jnp.concatenate([a_ref[0], a_ref[1]], axis=1)
    g2 = jnp.concatenate([g_ref[0], g_ref[1]], axis=1)
    out_ref[...] = jnp.maximum((agg + g2) * dis[:, None] + b_ref[...], 0.0)


def _t5_call(agg2, g2, dpart, b2):
    return pl.pallas_call(
        _t5_body,
        grid=(14,),
        in_specs=[
            pl.BlockSpec((2, RB, DH), lambda i: (0, i, 0)),
            pl.BlockSpec((2, RB, DH), lambda i: (0, i, 0)),
            pl.BlockSpec((2, RB), lambda i: (0, i)),
            pl.BlockSpec((1, D), lambda i: (0, 0)),
        ],
        out_specs=pl.BlockSpec((RB, D), lambda i: (i, 0)),
        out_shape=jax.ShapeDtypeStruct((NPAD, D), _f32),
    )(agg2, g2, dpart, b2)


def _t6_body(p_ref, c_ref, w_ref, b_ref, out_ref):
    sums = p_ref[0, :G, :] + p_ref[1, :G, :]
    cnt = c_ref[0, :G] + c_ref[1, :G]
    pooled = sums / jnp.maximum(cnt, 1.0)[:, None]
    out_ref[...] = lax.dot_general(pooled, w_ref[...],
                                   (((1,), (1,)), ((), ())),
                                   preferred_element_type=_f32) + b_ref[...]


def _t6_call(ppart, cpart, linW, linb):
    return pl.pallas_call(
        _t6_body,
        out_shape=jax.ShapeDtypeStruct((G, NCLS), _f32),
    )(ppart, cpart, linW, linb)


@jax.jit
def kernel(x, edge_index, batch, embed, W1, b1, W2, b2, linW, linb):
    x = x.astype(_i32)
    src = edge_index[0].astype(_i32)
    dst = edge_index[1].astype(_i32)
    batch = batch.astype(_i32)

    x_pad = jnp.concatenate([x, jnp.zeros((NPAD - N,), _i32)])
    src_pad = jnp.concatenate([src, jnp.zeros((EPAD - E,), _i32)])
    dst_pad = jnp.concatenate([dst, jnp.full((EPAD - E,), N, _i32)])
    batch_pad = jnp.concatenate([batch, jnp.full((NPAD - N,), G, _i32)])

    t1 = _t1_call(embed, W1)
    dego, cnto, hhat1 = _prep(dst_pad.reshape(2, NS, NCH0, 128),
                              batch_pad.reshape(2, NS, PT_N // 112, 112),
                              t1, x_pad)
    dpart = dego.reshape(2, NPAD)
    g1 = _t2_call(hhat1, dpart)

    src4 = src_pad.reshape(NS, NSLAB, SLAB, 128)
    dst4 = dst_pad.reshape(NS, NSLAB, SLAB, 128)
    agg1 = _agg(g1, src4, dst4).reshape(2, NPAD, DH)
    g2 = _t4_call(agg1, g1, dpart, W2, b1.reshape(1, D))

    agg2 = _agg(g2, src4, dst4).reshape(2, NPAD, DH)
    h2 = _t5_call(agg2, g2, dpart, b2.reshape(1, D))

    ppart = _pool(h2, batch_pad)
    return _t6_call(ppart.reshape(2, CAP, D), cnto.reshape(2, CAP),
                    linW, linb.reshape(1, NCLS))


# speedup vs baseline: 1.0182x; 1.0182x over previous
import functools

import jax
import jax.numpy as jnp
from jax import lax
from jax.experimental import pallas as pl
from jax.experimental.pallas import tpu as pltpu
from jax.experimental.pallas import tpu_sc as plsc

_f32 = jnp.float32
_i32 = jnp.int32

N = 50000
E = 800000
VOC = 10000
D = 64
DH = 32
G = 256
NCLS = 2

NC = 2
NS = 16
NW = NC * NS

NPAD = 50176
PT_N = NPAD // NW
EPAD = 811008
PT_E = EPAD // NS
PT_E0 = EPAD // NW
CAP = 264
RPT = NPAD // NS
EC = 96
NCH = PT_E // EC
GK = 4
SLAB = 16
NSLAB = NCH // SLAB
NCH0 = PT_E0 // 128

_mesh = plsc.VectorSubcoreMesh(core_axis_name="c", subcore_axis_name="s")
_sc_params = pltpu.CompilerParams(use_tc_tiling_on_sc=False)


def _fill_vec(ref, n16, value):
    def body(i, carry):
        ref[pl.ds(i * 16, 16)] = jnp.full((16,), value, _f32)
        return carry
    lax.fori_loop(0, n16, body, 0)


def _fill_rows(ref, nrows, ncol, value):
    def body(i, carry):
        for k in range(ncol // 16):
            ref[i, pl.ds(k * 16, 16)] = jnp.full((16,), value, _f32)
        return carry
    lax.fori_loop(0, nrows, body, 0)


@functools.partial(
    pl.kernel,
    out_type=(jax.ShapeDtypeStruct((2 * NPAD,), _f32),
              jax.ShapeDtypeStruct((2 * CAP,), _f32)),
    mesh=_mesh,
    compiler_params=_sc_params,
    scratch_types=(
        pltpu.VMEM((NCH0, 128), _i32),
        pltpu.VMEM((PT_N // 112, 112), _i32),
        pltpu.VMEM((128,), _f32),
        pltpu.VMEM((112,), _f32),
        pltpu.VMEM((112,), _f32),
        pltpu.VMEM_SHARED((NPAD,), _f32),
        pltpu.VMEM_SHARED((CAP,), _f32),
    ),
)
def _counts(dst_hbm, batch_hbm, dego_hbm, cnto_hbm,
            dstall, ball, ones_v, zero_v, bounce, dacc, cacc):
    c = lax.axis_index("c")
    s = lax.axis_index("s")
    _fill_vec(ones_v, 8, 1.0)
    _fill_vec(zero_v, 7, 0.0)
    pltpu.sync_copy(dst_hbm.at[c, s], dstall)
    pltpu.sync_copy(batch_hbm.at[c, s], ball)

    def zdeg(i, carry):
        pltpu.sync_copy(zero_v, dacc.at[pl.ds(s * RPT + i * 112, 112)])
        return carry
    lax.fori_loop(0, RPT // 112, zdeg, 0)

    @pl.when(s == 0)
    def _():
        pltpu.sync_copy(zero_v, cacc.at[pl.ds(0, 112)])
        pltpu.sync_copy(zero_v, cacc.at[pl.ds(112, 112)])
        pltpu.sync_copy(zero_v.at[pl.ds(0, 40)], cacc.at[pl.ds(224, 40)])

    plsc.subcore_barrier()

    def edges(j, carry):
        pltpu.sync_copy(ones_v, dacc.at[dstall.at[j]], add=True)
        return carry
    lax.fori_loop(0, NCH0, edges, 0)

    def nodes(j, carry):
        pltpu.sync_copy(ones_v.at[pl.ds(0, 112)], cacc.at[ball.at[j]], add=True)
        return carry
    lax.fori_loop(0, PT_N // 112, nodes, 0)

    plsc.subcore_barrier()

    def wdeg(i, carry):
        r0 = s * RPT + i * 112
        pltpu.sync_copy(dacc.at[pl.ds(r0, 112)], bounce)
        pltpu.sync_copy(bounce, dego_hbm.at[pl.ds(c * NPAD + r0, 112)])
        return carry
    lax.fori_loop(0, RPT // 112, wdeg, 0)

    @pl.when(s == 0)
    def _():
        pltpu.sync_copy(cacc.at[pl.ds(0, 112)], bounce)
        pltpu.sync_copy(bounce, cnto_hbm.at[pl.ds(c * CAP, 112)])
        pltpu.sync_copy(cacc.at[pl.ds(112, 112)], bounce)
        pltpu.sync_copy(bounce, cnto_hbm.at[pl.ds(c * CAP + 112, 112)])
        pltpu.sync_copy(cacc.at[pl.ds(224, 40)], bounce.at[pl.ds(0, 40)])
        pltpu.sync_copy(bounce.at[pl.ds(0, 40)],
                        cnto_hbm.at[pl.ds(c * CAP + 224, 40)])


@functools.partial(
    pl.kernel,
    out_type=jax.ShapeDtypeStruct((NPAD, D), _f32),
    mesh=_mesh,
    compiler_params=_sc_params,
    scratch_types=(
        pltpu.VMEM((112,), _i32),
        pltpu.VMEM((112, D), _f32),
        pltpu.SemaphoreType.DMA,
    ),
)
def _gather(t1_hbm, x_hbm, out_hbm, xidx, rows, sem):
    c = lax.axis_index("c")
    s = lax.axis_index("s")
    wid = s * NC + c

    def body(j, carry):
        base = wid * PT_N + j * 112
        pltpu.sync_copy(x_hbm.at[pl.ds(base, 112)], xidx)
        pltpu.async_copy(t1_hbm.at[xidx], rows, sem).wait()
        pltpu.sync_copy(rows, out_hbm.at[pl.ds(base, 112)])
        return carry
    lax.fori_loop(0, PT_N // 112, body, 0)


@functools.partial(
    pl.kernel,
    out_type=jax.ShapeDtypeStruct((2 * NPAD, DH), _f32),
    mesh=_mesh,
    compiler_params=_sc_params,
    scratch_types=(
        pltpu.VMEM((SLAB, EC), _i32),
        pltpu.VMEM((SLAB, EC), _i32),
        pltpu.VMEM((2 * GK * EC, DH), _f32),
        pltpu.VMEM_SHARED((NPAD, DH), _f32),
        pltpu.SemaphoreType.DMA,
        pltpu.SemaphoreType.DMA,
    ),
)
def _agg(g_hbm, srcoff_hbm, dst_hbm, out_hbm,
         sidx, didx, rows, acc, sem, ssem):
    c = lax.axis_index("c")
    s = lax.axis_index("s")
    _fill_rows(rows, 112, DH, 0.0)
    zrow = rows.at[pl.ds(0, 112)]

    def zacc(i, carry):
        pltpu.sync_copy(zrow, acc.at[pl.ds(s * RPT + i * 112, 112)])
        return carry
    lax.fori_loop(0, RPT // 112, zacc, 0)

    plsc.subcore_barrier()

    def buf(k):
        return rows.at[pl.ds(k * EC, EC)]

    def fire(grp, par):
        for b in range(GK):
            pltpu.async_copy(g_hbm.at[sidx.at[grp * GK + b]],
                             buf(par * GK + b), sem)

    NGRP = SLAB // GK

    def drain_scatter(par):
        for b in range(GK):
            pltpu.make_async_copy(buf(par * GK + b),
                                  acc.at[pl.ds(0, EC)], ssem).wait()

    for sl in range(NSLAB):
        pltpu.sync_copy(srcoff_hbm.at[c, s, sl], sidx)
        pltpu.sync_copy(dst_hbm.at[s, sl], didx)
        fire(0, 0)

        def dbl(t, carry):
            for par in (0, 1):
                grp = t * 2 + par
                for b in range(GK):
                    pltpu.make_async_copy(g_hbm.at[pl.ds(0, EC)],
                                          buf(par * GK + b), sem).wait()

                @pl.when(grp >= 1)
                def _():
                    drain_scatter(1 - par)

                @pl.when(grp + 1 < NGRP)
                def _():
                    fire(grp + 1, 1 - par)

                for b in range(GK):
                    pltpu.async_copy(buf(par * GK + b),
                                     acc.at[didx.at[grp * GK + b]], ssem,
                                     add=True)
            return carry
        lax.fori_loop(0, NGRP // 2, dbl, 0)
        drain_scatter(1)

    plsc.subcore_barrier()

    def wout(i, carry):
        r0 = s * RPT + i * 112
        zb = rows.at[pl.ds(0, 112)]
        pltpu.sync_copy(acc.at[pl.ds(r0, 112)], zb)
        pltpu.sync_copy(zb, out_hbm.at[pl.ds(c * NPAD + r0, 112)])
        return carry
    lax.fori_loop(0, RPT // 112, wout, 0)


@functools.partial(
    pl.kernel,
    out_type=jax.ShapeDtypeStruct((2 * CAP, D), _f32),
    mesh=_mesh,
    compiler_params=_sc_params,
    scratch_types=(
        pltpu.VMEM((112,), _i32),
        pltpu.VMEM((112, D), _f32),
        pltpu.VMEM((112, D), _f32),
        pltpu.VMEM_SHARED((CAP, D), _f32),
    ),
)
def _pool(h2_hbm, batch_hbm, out_hbm, bidx, rows, zrow, pacc):
    c = lax.axis_index("c")
    s = lax.axis_index("s")
    _fill_rows(zrow, 112, D, 0.0)

    @pl.when(s == 0)
    def _():
        pltpu.sync_copy(zrow, pacc.at[pl.ds(0, 112)])
        pltpu.sync_copy(zrow, pacc.at[pl.ds(112, 112)])
        pltpu.sync_copy(zrow.at[pl.ds(0, 40)], pacc.at[pl.ds(224, 40)])

    plsc.subcore_barrier()

    def body(j, carry):
        base = c * (NPAD // 2) + s * PT_N + j * 112
        pltpu.sync_copy(batch_hbm.at[pl.ds(base, 112)], bidx)
        pltpu.sync_copy(h2_hbm.at[pl.ds(base, 112)], rows)
        pltpu.sync_copy(rows, pacc.at[bidx], add=True)
        return carry
    lax.fori_loop(0, PT_N // 112, body, 0)

    plsc.subcore_barrier()

    @pl.when(s == 0)
    def _():
        pltpu.sync_copy(pacc.at[pl.ds(0, 112)], zrow)
        pltpu.sync_copy(zrow, out_hbm.at[pl.ds(c * CAP, 112)])
        pltpu.sync_copy(pacc.at[pl.ds(112, 112)], zrow)
        pltpu.sync_copy(zrow, out_hbm.at[pl.ds(c * CAP + 112, 112)])
        pltpu.sync_copy(pacc.at[pl.ds(224, 40)], zrow.at[pl.ds(0, 40)])
        pltpu.sync_copy(zrow.at[pl.ds(0, 40)],
                        out_hbm.at[pl.ds(c * CAP + 224, 40)])


RB = NPAD // 14


def _t1_body(emb_ref, w_ref, out_ref):
    out_ref[...] = lax.dot_general(emb_ref[...], w_ref[...],
                                   (((1,), (1,)), ((), ())),
                                   preferred_element_type=_f32)


def _t1_call(embed, W1):
    return pl.pallas_call(
        _t1_body,
        out_shape=jax.ShapeDtypeStruct((VOC, D), _f32),
    )(embed, W1)


def _t2_body(h_ref, d_ref, out_ref):
    dis = lax.rsqrt(d_ref[0] + d_ref[1] + 1.0)
    g = h_ref[...] * dis[:, None]
    out_ref[0] = g[:, :DH]
    out_ref[1] = g[:, DH:]


def _t2_call(hhat1, dpart):
    return pl.pallas_call(
        _t2_body,
        grid=(14,),
        in_specs=[
            pl.BlockSpec((RB, D), lambda i: (i, 0)),
            pl.BlockSpec((2, RB), lambda i: (0, i)),
        ],
        out_specs=pl.BlockSpec((2, RB, DH), lambda i: (0, i, 0)),
        out_shape=jax.ShapeDtypeStruct((2, NPAD, DH), _f32),
    )(hhat1, dpart)


def _t4_body(a_ref, g_ref, d_ref, w_ref, b_ref, out_ref):
    dis = lax.rsqrt(d_ref[0] + d_ref[1] + 1.0)
    agg = jnp.concatenate([a_ref[0], a_ref[1]], axis=1)
    g1 = jnp.concatenate([g_ref[0], g_ref[1]], axis=1)
    h1 = jnp.maximum((agg + g1) * dis[:, None] + b_ref[...], 0.0)
    hh2 = lax.dot_general(h1, w_ref[...], (((1,), (1,)), ((), ())),
                          preferred_element_type=_f32)
    g2 = hh2 * dis[:, None]
    out_ref[0] = g2[:, :DH]
    out_ref[1] = g2[:, DH:]


def _t4_call(agg1, g1, dpart, W2, b1):
    return pl.pallas_call(
        _t4_body,
        grid=(14,),
        in_specs=[
            pl.BlockSpec((2, RB, DH), lambda i: (0, i, 0)),
            pl.BlockSpec((2, RB, DH), lambda i: (0, i, 0)),
            pl.BlockSpec((2, RB), lambda i: (0, i)),
            pl.BlockSpec((D, D), lambda i: (0, 0)),
            pl.BlockSpec((1, D), lambda i: (0, 0)),
        ],
        out_specs=pl.BlockSpec((2, RB, DH), lambda i: (0, i, 0)),
        out_shape=jax.ShapeDtypeStruct((2, NPAD, DH), _f32),
    )(agg1, g1, dpart, W2, b1)


def _t5_body(a_ref, g_ref, d_ref, b_ref, out_ref):
    dis = lax.rsqrt(d_ref[0] + d_ref[1] + 1.0)
    agg = jnp.concatenate([a_ref[0], a_ref[1]], axis=1)
    g2 = jnp.concatenate([g_ref[0], g_ref[1]], axis=1)
    out_ref[...] = jnp.maximum((agg + g2) * dis[:, None] + b_ref[...], 0.0)


def _t5_call(agg2, g2, dpart, b2):
    return pl.pallas_call(
        _t5_body,
        grid=(14,),
        in_specs=[
            pl.BlockSpec((2, RB, DH), lambda i: (0, i, 0)),
            pl.BlockSpec((2, RB, DH), lambda i: (0, i, 0)),
            pl.BlockSpec((2, RB), lambda i: (0, i)),
            pl.BlockSpec((1, D), lambda i: (0, 0)),
        ],
        out_specs=pl.BlockSpec((RB, D), lambda i: (i, 0)),
        out_shape=jax.ShapeDtypeStruct((NPAD, D), _f32),
    )(agg2, g2, dpart, b2)


def _t6_body(p_ref, c_ref, w_ref, b_ref, out_ref):
    sums = p_ref[0, :G, :] + p_ref[1, :G, :]
    cnt = c_ref[0, :G] + c_ref[1, :G]
    pooled = sums / jnp.maximum(cnt, 1.0)[:, None]
    out_ref[...] = lax.dot_general(pooled, w_ref[...],
                                   (((1,), (1,)), ((), ())),
                                   preferred_element_type=_f32) + b_ref[...]


def _t6_call(ppart, cpart, linW, linb):
    return pl.pallas_call(
        _t6_body,
        out_shape=jax.ShapeDtypeStruct((G, NCLS), _f32),
    )(ppart, cpart, linW, linb)


@jax.jit
def kernel(x, edge_index, batch, embed, W1, b1, W2, b2, linW, linb):
    x = x.astype(_i32)
    src = edge_index[0].astype(_i32)
    dst = edge_index[1].astype(_i32)
    batch = batch.astype(_i32)

    x_pad = jnp.concatenate([x, jnp.zeros((NPAD - N,), _i32)])
    src_pad = jnp.concatenate([src, jnp.zeros((EPAD - E,), _i32)])
    dst_pad = jnp.concatenate([dst, jnp.full((EPAD - E,), N, _i32)])
    srcoff = jnp.concatenate([src_pad, src_pad + NPAD])
    batch_pad = jnp.concatenate([batch, jnp.full((NPAD - N,), G, _i32)])

    dego, cnto = _counts(dst_pad.reshape(2, NS, NCH0, 128),
                         batch_pad.reshape(2, NS, PT_N // 112, 112))
    dpart = dego.reshape(2, NPAD)

    t1 = _t1_call(embed, W1)
    hhat1 = _gather(t1, x_pad)
    g1 = _t2_call(hhat1, dpart)

    srcoff4 = srcoff.reshape(2, NS, NSLAB, SLAB, EC)
    dst4 = dst_pad.reshape(NS, NSLAB, SLAB, EC)
    agg1 = _agg(g1.reshape(2 * NPAD, DH), srcoff4, dst4).reshape(2, NPAD, DH)
    g2 = _t4_call(agg1, g1, dpart, W2, b1.reshape(1, D))

    agg2 = _agg(g2.reshape(2 * NPAD, DH), srcoff4, dst4).reshape(2, NPAD, DH)
    h2 = _t5_call(agg2, g2, dpart, b2.reshape(1, D))

    ppart = _pool(h2, batch_pad)
    return _t6_call(ppart.reshape(2, CAP, D), cnto.reshape(2, CAP),
                    linW, linb.reshape(1, NCLS))


# speedup vs baseline: 1.1618x; 1.1411x over previous
import functools

import jax
import jax.numpy as jnp
from jax import lax
from jax.experimental import pallas as pl
from jax.experimental.pallas import tpu as pltpu
from jax.experimental.pallas import tpu_sc as plsc

_f32 = jnp.float32
_i32 = jnp.int32

N = 50000
E = 800000
VOC = 10000
D = 64
DH = 32
G = 256
NCLS = 2

NC = 2
NS = 16
NW = NC * NS

NPAD = 50176
PT_N = NPAD // NW
EPAD = 806400
PT_E = EPAD // NS
PT_E0 = EPAD // NW
CAP = 264
RPT = NPAD // NS
EC = 112
NCH = PT_E // EC
GK = 3
SLAB = 18
NSLAB = NCH // SLAB
NCH0 = PT_E0 // EC

_mesh = plsc.VectorSubcoreMesh(core_axis_name="c", subcore_axis_name="s")
_sc_params = pltpu.CompilerParams(use_tc_tiling_on_sc=False)


def _fill_vec(ref, n16, value):
    def body(i, carry):
        ref[pl.ds(i * 16, 16)] = jnp.full((16,), value, _f32)
        return carry
    lax.fori_loop(0, n16, body, 0)


def _fill_rows(ref, nrows, ncol, value):
    def body(i, carry):
        for k in range(ncol // 16):
            ref[i, pl.ds(k * 16, 16)] = jnp.full((16,), value, _f32)
        return carry
    lax.fori_loop(0, nrows, body, 0)


@functools.partial(
    pl.kernel,
    out_type=(jax.ShapeDtypeStruct((2 * NPAD,), _f32),
              jax.ShapeDtypeStruct((2 * CAP,), _f32)),
    mesh=_mesh,
    compiler_params=_sc_params,
    scratch_types=(
        pltpu.VMEM((NCH0, EC), _i32),
        pltpu.VMEM((PT_N // 112, 112), _i32),
        pltpu.VMEM((128,), _f32),
        pltpu.VMEM((112,), _f32),
        pltpu.VMEM((112,), _f32),
        pltpu.VMEM_SHARED((NPAD,), _f32),
        pltpu.VMEM_SHARED((CAP,), _f32),
    ),
)
def _counts(dst_hbm, batch_hbm, dego_hbm, cnto_hbm,
            dstall, ball, ones_v, zero_v, bounce, dacc, cacc):
    c = lax.axis_index("c")
    s = lax.axis_index("s")
    _fill_vec(ones_v, 8, 1.0)
    _fill_vec(zero_v, 7, 0.0)
    pltpu.sync_copy(dst_hbm.at[c, s], dstall)
    pltpu.sync_copy(batch_hbm.at[c, s], ball)

    def zdeg(i, carry):
        pltpu.sync_copy(zero_v, dacc.at[pl.ds(s * RPT + i * 112, 112)])
        return carry
    lax.fori_loop(0, RPT // 112, zdeg, 0)

    @pl.when(s == 0)
    def _():
        pltpu.sync_copy(zero_v, cacc.at[pl.ds(0, 112)])
        pltpu.sync_copy(zero_v, cacc.at[pl.ds(112, 112)])
        pltpu.sync_copy(zero_v.at[pl.ds(0, 40)], cacc.at[pl.ds(224, 40)])

    plsc.subcore_barrier()

    def edges(j, carry):
        pltpu.sync_copy(ones_v.at[pl.ds(0, EC)], dacc.at[dstall.at[j]],
                        add=True)
        return carry
    lax.fori_loop(0, NCH0, edges, 0)

    def nodes(j, carry):
        pltpu.sync_copy(ones_v.at[pl.ds(0, 112)], cacc.at[ball.at[j]], add=True)
        return carry
    lax.fori_loop(0, PT_N // 112, nodes, 0)

    plsc.subcore_barrier()

    def wdeg(i, carry):
        r0 = s * RPT + i * 112
        pltpu.sync_copy(dacc.at[pl.ds(r0, 112)], bounce)
        pltpu.sync_copy(bounce, dego_hbm.at[pl.ds(c * NPAD + r0, 112)])
        return carry
    lax.fori_loop(0, RPT // 112, wdeg, 0)

    @pl.when(s == 0)
    def _():
        pltpu.sync_copy(cacc.at[pl.ds(0, 112)], bounce)
        pltpu.sync_copy(bounce, cnto_hbm.at[pl.ds(c * CAP, 112)])
        pltpu.sync_copy(cacc.at[pl.ds(112, 112)], bounce)
        pltpu.sync_copy(bounce, cnto_hbm.at[pl.ds(c * CAP + 112, 112)])
        pltpu.sync_copy(cacc.at[pl.ds(224, 40)], bounce.at[pl.ds(0, 40)])
        pltpu.sync_copy(bounce.at[pl.ds(0, 40)],
                        cnto_hbm.at[pl.ds(c * CAP + 224, 40)])


@functools.partial(
    pl.kernel,
    out_type=jax.ShapeDtypeStruct((NPAD, D), _f32),
    mesh=_mesh,
    compiler_params=_sc_params,
    scratch_types=(
        pltpu.VMEM((PT_N // 112, 112), _i32),
        pltpu.VMEM((2, 112, D), _f32),
        pltpu.SemaphoreType.DMA,
        pltpu.SemaphoreType.DMA,
    ),
)
def _gather(t1_hbm, x_hbm, out_hbm, xall, rows, sem, wsem):
    c = lax.axis_index("c")
    s = lax.axis_index("s")
    wid = s * NC + c
    NJ = PT_N // 112
    pltpu.sync_copy(x_hbm.at[wid], xall)
    pltpu.async_copy(t1_hbm.at[xall.at[0]], rows.at[0], sem)

    def body(t, carry):
        for p in (0, 1):
            jj = 2 * t + p
            pltpu.make_async_copy(t1_hbm.at[xall.at[0]], rows.at[p],
                                  sem).wait()

            @pl.when(jj >= 1)
            def _():
                pltpu.make_async_copy(rows.at[1 - p],
                                      out_hbm.at[pl.ds(0, 112)], wsem).wait()

            @pl.when(jj + 1 < NJ)
            def _():
                pltpu.async_copy(t1_hbm.at[xall.at[jj + 1]], rows.at[1 - p],
                                 sem)

            pltpu.async_copy(rows.at[p],
                             out_hbm.at[pl.ds(wid * PT_N + jj * 112, 112)],
                             wsem)
        return carry
    lax.fori_loop(0, NJ // 2, body, 0)
    pltpu.make_async_copy(rows.at[1], out_hbm.at[pl.ds(0, 112)], wsem).wait()


@functools.partial(
    pl.kernel,
    out_type=jax.ShapeDtypeStruct((2 * NPAD, DH), _f32),
    mesh=_mesh,
    compiler_params=_sc_params,
    scratch_types=(
        pltpu.VMEM((2, SLAB, EC), _i32),
        pltpu.VMEM((2, SLAB, EC), _i32),
        pltpu.VMEM((2 * GK * EC, DH), _f32),
        pltpu.VMEM_SHARED((NPAD, DH), _f32),
        pltpu.SemaphoreType.DMA,
        pltpu.SemaphoreType.DMA,
        pltpu.SemaphoreType.DMA,
    ),
)
def _agg(g_hbm, srcoff_hbm, dst_hbm, out_hbm,
         sidx, didx, rows, acc, sem, ssem, isem):
    c = lax.axis_index("c")
    s = lax.axis_index("s")
    _fill_rows(rows, 112, DH, 0.0)
    zrow = rows.at[pl.ds(0, 112)]

    def zacc(i, carry):
        pltpu.sync_copy(zrow, acc.at[pl.ds(s * RPT + i * 112, 112)])
        return carry
    lax.fori_loop(0, RPT // 112, zacc, 0)

    plsc.subcore_barrier()

    def buf(k):
        return rows.at[pl.ds(k * EC, EC)]

    def fire(grp, par, q):
        for b in range(GK):
            pltpu.async_copy(g_hbm.at[sidx.at[q].at[grp * GK + b]],
                             buf(par * GK + b), sem)

    NGRP = SLAB // GK

    def drain_scatter(par):
        for b in range(GK):
            pltpu.make_async_copy(buf(par * GK + b),
                                  acc.at[pl.ds(0, EC)], ssem).wait()

    def slab_load(sl, q):
        pltpu.async_copy(srcoff_hbm.at[c, s, sl], sidx.at[q], isem)
        pltpu.async_copy(dst_hbm.at[s, sl], didx.at[q], isem)

    def slab_wait():
        pltpu.make_async_copy(srcoff_hbm.at[c, s, 0], sidx.at[0], isem).wait()
        pltpu.make_async_copy(dst_hbm.at[s, 0], didx.at[0], isem).wait()

    def do_slab(q, nxt):
        slab_wait()
        if nxt is not None:
            slab_load(nxt, 1 - q)
        fire(0, 0, q)

        def dbl(t, carry):
            for par in (0, 1):
                grp = t * 2 + par
                for b in range(GK):
                    pltpu.make_async_copy(g_hbm.at[pl.ds(0, EC)],
                                          buf(par * GK + b), sem).wait()

                @pl.when(grp >= 1)
                def _():
                    drain_scatter(1 - par)

                @pl.when(grp + 1 < NGRP)
                def _():
                    fire(grp + 1, 1 - par, q)

                for b in range(GK):
                    pltpu.async_copy(buf(par * GK + b),
                                     acc.at[didx.at[q].at[grp * GK + b]], ssem,
                                     add=True)
            return carry
        lax.fori_loop(0, NGRP // 2, dbl, 0)
        drain_scatter(1)

    slab_load(0, 0)

    def dslab(t, carry):
        do_slab(0, 2 * t + 1)
        do_slab(1, 2 * t + 2)
        return carry
    lax.fori_loop(0, (NSLAB - 1) // 2, dslab, 0)
    do_slab(0, None)

    plsc.subcore_barrier()

    def wout(i, carry):
        r0 = s * RPT + i * 112
        zb = rows.at[pl.ds(0, 112)]
        pltpu.sync_copy(acc.at[pl.ds(r0, 112)], zb)
        pltpu.sync_copy(zb, out_hbm.at[pl.ds(c * NPAD + r0, 112)])
        return carry
    lax.fori_loop(0, RPT // 112, wout, 0)


@functools.partial(
    pl.kernel,
    out_type=jax.ShapeDtypeStruct((2 * CAP, D), _f32),
    mesh=_mesh,
    compiler_params=_sc_params,
    scratch_types=(
        pltpu.VMEM((PT_N // 112, 112), _i32),
        pltpu.VMEM((2, 112, D), _f32),
        pltpu.VMEM((112, D), _f32),
        pltpu.VMEM_SHARED((CAP, D), _f32),
        pltpu.SemaphoreType.DMA,
        pltpu.SemaphoreType.DMA,
    ),
)
def _pool(h2_hbm, batch_hbm, out_hbm, ball, rows, zrow, pacc, sem, ssem):
    c = lax.axis_index("c")
    s = lax.axis_index("s")
    NJ = PT_N // 112
    base0 = c * (NPAD // 2) + s * PT_N
    _fill_rows(zrow, 112, D, 0.0)
    pltpu.sync_copy(batch_hbm.at[c, s], ball)

    @pl.when(s == 0)
    def _():
        pltpu.sync_copy(zrow, pacc.at[pl.ds(0, 112)])
        pltpu.sync_copy(zrow, pacc.at[pl.ds(112, 112)])
        pltpu.sync_copy(zrow.at[pl.ds(0, 40)], pacc.at[pl.ds(224, 40)])

    plsc.subcore_barrier()

    pltpu.async_copy(h2_hbm.at[pl.ds(base0, 112)], rows.at[0], sem)

    def body(t, carry):
        for p in (0, 1):
            jj = 2 * t + p
            pltpu.make_async_copy(h2_hbm.at[pl.ds(0, 112)], rows.at[p],
                                  sem).wait()

            @pl.when(jj >= 1)
            def _():
                pltpu.make_async_copy(rows.at[1 - p], pacc.at[pl.ds(0, 112)],
                                      ssem).wait()

            @pl.when(jj + 1 < NJ)
            def _():
                pltpu.async_copy(
                    h2_hbm.at[pl.ds(base0 + (jj + 1) * 112, 112)],
                    rows.at[1 - p], sem)

            pltpu.async_copy(rows.at[p], pacc.at[ball.at[jj]], ssem, add=True)
        return carry
    lax.fori_loop(0, NJ // 2, body, 0)
    pltpu.make_async_copy(rows.at[1], pacc.at[pl.ds(0, 112)], ssem).wait()

    plsc.subcore_barrier()

    @pl.when(s == 0)
    def _():
        pltpu.sync_copy(pacc.at[pl.ds(0, 112)], zrow)
        pltpu.sync_copy(zrow, out_hbm.at[pl.ds(c * CAP, 112)])
        pltpu.sync_copy(pacc.at[pl.ds(112, 112)], zrow)
        pltpu.sync_copy(zrow, out_hbm.at[pl.ds(c * CAP + 112, 112)])
        pltpu.sync_copy(pacc.at[pl.ds(224, 40)], zrow.at[pl.ds(0, 40)])
        pltpu.sync_copy(zrow.at[pl.ds(0, 40)],
                        out_hbm.at[pl.ds(c * CAP + 224, 40)])


RB = NPAD // 14


def _t1_body(emb_ref, w_ref, out_ref):
    out_ref[...] = lax.dot_general(emb_ref[...], w_ref[...],
                                   (((1,), (1,)), ((), ())),
                                   preferred_element_type=_f32)


def _t1_call(embed, W1):
    return pl.pallas_call(
        _t1_body,
        out_shape=jax.ShapeDtypeStruct((VOC, D), _f32),
    )(embed, W1)


def _t2_body(h_ref, d_ref, out_ref):
    dis = lax.rsqrt(d_ref[0] + d_ref[1] + 1.0)
    g = h_ref[...] * dis[:, None]
    out_ref[0] = g[:, :DH]
    out_ref[1] = g[:, DH:]


def _t2_call(hhat1, dpart):
    return pl.pallas_call(
        _t2_body,
        grid=(14,),
        in_specs=[
            pl.BlockSpec((RB, D), lambda i: (i, 0)),
            pl.BlockSpec((2, RB), lambda i: (0, i)),
        ],
        out_specs=pl.BlockSpec((2, RB, DH), lambda i: (0, i, 0)),
        out_shape=jax.ShapeDtypeStruct((2, NPAD, DH), _f32),
    )(hhat1, dpart)


def _t4_body(a_ref, g_ref, d_ref, w_ref, b_ref, out_ref):
    dis = lax.rsqrt(d_ref[0] + d_ref[1] + 1.0)
    agg = jnp.concatenate([a_ref[0], a_ref[1]], axis=1)
    g1 = jnp.concatenate([g_ref[0], g_ref[1]], axis=1)
    h1 = jnp.maximum((agg + g1) * dis[:, None] + b_ref[...], 0.0)
    hh2 = lax.dot_general(h1, w_ref[...], (((1,), (1,)), ((), ())),
                          preferred_element_type=_f32)
    g2 = hh2 * dis[:, None]
    out_ref[0] = g2[:, :DH]
    out_ref[1] = g2[:, DH:]


def _t4_call(agg1, g1, dpart, W2, b1):
    return pl.pallas_call(
        _t4_body,
        grid=(14,),
        in_specs=[
            pl.BlockSpec((2, RB, DH), lambda i: (0, i, 0)),
            pl.BlockSpec((2, RB, DH), lambda i: (0, i, 0)),
            pl.BlockSpec((2, RB), lambda i: (0, i)),
            pl.BlockSpec((D, D), lambda i: (0, 0)),
            pl.BlockSpec((1, D), lambda i: (0, 0)),
        ],
        out_specs=pl.BlockSpec((2, RB, DH), lambda i: (0, i, 0)),
        out_shape=jax.ShapeDtypeStruct((2, NPAD, DH), _f32),
    )(agg1, g1, dpart, W2, b1)


def _t5_body(a_ref, g_ref, d_ref, b_ref, out_ref):
    dis = lax.rsqrt(d_ref[0] + d_ref[1] + 1.0)
    agg = jnp.concatenate([a_ref[0], a_ref[1]], axis=1)
    g2 = jnp.concatenate([g_ref[0], g_ref[1]], axis=1)
    out_ref[...] = jnp.maximum((agg + g2) * dis[:, None] + b_ref[...], 0.0)


def _t5_call(agg2, g2, dpart, b2):
    return pl.pallas_call(
        _t5_body,
        grid=(14,),
        in_specs=[
            pl.BlockSpec((2, RB, DH), lambda i: (0, i, 0)),
            pl.BlockSpec((2, RB, DH), lambda i: (0, i, 0)),
            pl.BlockSpec((2, RB), lambda i: (0, i)),
            pl.BlockSpec((1, D), lambda i: (0, 0)),
        ],
        out_specs=pl.BlockSpec((RB, D), lambda i: (i, 0)),
        out_shape=jax.ShapeDtypeStruct((NPAD, D), _f32),
    )(agg2, g2, dpart, b2)


def _t6_body(p_ref, c_ref, w_ref, b_ref, out_ref):
    sums = p_ref[0, :G, :] + p_ref[1, :G, :]
    cnt = c_ref[0, :G] + c_ref[1, :G]
    pooled = sums / jnp.maximum(cnt, 1.0)[:, None]
    out_ref[...] = lax.dot_general(pooled, w_ref[...],
                                   (((1,), (1,)), ((), ())),
                                   preferred_element_type=_f32) + b_ref[...]


def _t6_call(ppart, cpart, linW, linb):
    return pl.pallas_call(
        _t6_body,
        out_shape=jax.ShapeDtypeStruct((G, NCLS), _f32),
    )(ppart, cpart, linW, linb)


@jax.jit
def kernel(x, edge_index, batch, embed, W1, b1, W2, b2, linW, linb):
    x = x.astype(_i32)
    src = edge_index[0].astype(_i32)
    dst = edge_index[1].astype(_i32)
    batch = batch.astype(_i32)

    x_pad = jnp.concatenate([x, jnp.zeros((NPAD - N,), _i32)])
    src_pad = jnp.concatenate([src, jnp.zeros((EPAD - E,), _i32)])
    dst_pad = jnp.concatenate([dst, jnp.full((EPAD - E,), N, _i32)])
    srcoff = jnp.concatenate([src_pad, src_pad + NPAD])
    batch_pad = jnp.concatenate([batch, jnp.full((NPAD - N,), G, _i32)])

    dego, cnto = _counts(dst_pad.reshape(2, NS, NCH0, EC),
                         batch_pad.reshape(2, NS, PT_N // 112, 112))
    dpart = dego.reshape(2, NPAD)

    t1 = _t1_call(embed, W1)
    hhat1 = _gather(t1, x_pad.reshape(NW, PT_N // 112, 112))
    g1 = _t2_call(hhat1, dpart)

    srcoff4 = srcoff.reshape(2, NS, NSLAB, SLAB, EC)
    dst4 = dst_pad.reshape(NS, NSLAB, SLAB, EC)
    agg1 = _agg(g1.reshape(2 * NPAD, DH), srcoff4, dst4).reshape(2, NPAD, DH)
    g2 = _t4_call(agg1, g1, dpart, W2, b1.reshape(1, D))

    agg2 = _agg(g2.reshape(2 * NPAD, DH), srcoff4, dst4).reshape(2, NPAD, DH)
    h2 = _t5_call(agg2, g2, dpart, b2.reshape(1, D))

    ppart = _pool(h2, batch_pad.reshape(2, NS, PT_N // 112, 112))
    return _t6_call(ppart.reshape(2, CAP, D), cnto.reshape(2, CAP),
                    linW, linb.reshape(1, NCLS))


# speedup vs baseline: 1.1748x; 1.0112x over previous
import functools

import jax
import jax.numpy as jnp
from jax import lax
from jax.experimental import pallas as pl
from jax.experimental.pallas import tpu as pltpu
from jax.experimental.pallas import tpu_sc as plsc

_f32 = jnp.float32
_i32 = jnp.int32

N = 50000
E = 800000
VOC = 10000
D = 64
DH = 32
G = 256
NCLS = 2

NC = 2
NS = 16
NW = NC * NS

NPAD = 50176
PT_N = NPAD // NW
EPAD = 806400
PT_E = EPAD // NS
PT_E0 = EPAD // NW
CAP = 264
RPT = NPAD // NS
EC = 112
NCH = PT_E // EC
GK = 3
SLAB = 18
NSLAB = NCH // SLAB
NCH0 = PT_E0 // EC

_mesh = plsc.VectorSubcoreMesh(core_axis_name="c", subcore_axis_name="s")
_sc_params = pltpu.CompilerParams(use_tc_tiling_on_sc=False)


def _fill_vec(ref, n16, value):
    def body(i, carry):
        ref[pl.ds(i * 16, 16)] = jnp.full((16,), value, _f32)
        return carry
    lax.fori_loop(0, n16, body, 0)


def _fill_rows(ref, nrows, ncol, value):
    def body(i, carry):
        for k in range(ncol // 16):
            ref[i, pl.ds(k * 16, 16)] = jnp.full((16,), value, _f32)
        return carry
    lax.fori_loop(0, nrows, body, 0)


@functools.partial(
    pl.kernel,
    out_type=(jax.ShapeDtypeStruct((2 * NPAD,), _f32),
              jax.ShapeDtypeStruct((2 * CAP,), _f32)),
    mesh=_mesh,
    compiler_params=_sc_params,
    scratch_types=(
        pltpu.VMEM((NCH0, EC), _i32),
        pltpu.VMEM((PT_N // 112, 112), _i32),
        pltpu.VMEM((128,), _f32),
        pltpu.VMEM((112,), _f32),
        pltpu.VMEM((112,), _f32),
        pltpu.VMEM_SHARED((NPAD,), _f32),
        pltpu.VMEM_SHARED((CAP,), _f32),
    ),
)
def _counts(dst_hbm, batch_hbm, dego_hbm, cnto_hbm,
            dstall, ball, ones_v, zero_v, bounce, dacc, cacc):
    c = lax.axis_index("c")
    s = lax.axis_index("s")
    _fill_vec(ones_v, 8, 1.0)
    _fill_vec(zero_v, 7, 0.0)
    pltpu.sync_copy(dst_hbm.at[c, s], dstall)
    pltpu.sync_copy(batch_hbm.at[c, s], ball)

    def zdeg(i, carry):
        pltpu.sync_copy(zero_v, dacc.at[pl.ds(s * RPT + i * 112, 112)])
        return carry
    lax.fori_loop(0, RPT // 112, zdeg, 0)

    @pl.when(s == 0)
    def _():
        pltpu.sync_copy(zero_v, cacc.at[pl.ds(0, 112)])
        pltpu.sync_copy(zero_v, cacc.at[pl.ds(112, 112)])
        pltpu.sync_copy(zero_v.at[pl.ds(0, 40)], cacc.at[pl.ds(224, 40)])

    plsc.subcore_barrier()

    def edges(j, carry):
        pltpu.sync_copy(ones_v.at[pl.ds(0, EC)], dacc.at[dstall.at[j]],
                        add=True)
        return carry
    lax.fori_loop(0, NCH0, edges, 0)

    def nodes(j, carry):
        pltpu.sync_copy(ones_v.at[pl.ds(0, 112)], cacc.at[ball.at[j]], add=True)
        return carry
    lax.fori_loop(0, PT_N // 112, nodes, 0)

    plsc.subcore_barrier()

    def wdeg(i, carry):
        r0 = s * RPT + i * 112
        pltpu.sync_copy(dacc.at[pl.ds(r0, 112)], bounce)
        pltpu.sync_copy(bounce, dego_hbm.at[pl.ds(c * NPAD + r0, 112)])
        return carry
    lax.fori_loop(0, RPT // 112, wdeg, 0)

    @pl.when(s == 0)
    def _():
        pltpu.sync_copy(cacc.at[pl.ds(0, 112)], bounce)
        pltpu.sync_copy(bounce, cnto_hbm.at[pl.ds(c * CAP, 112)])
        pltpu.sync_copy(cacc.at[pl.ds(112, 112)], bounce)
        pltpu.sync_copy(bounce, cnto_hbm.at[pl.ds(c * CAP + 112, 112)])
        pltpu.sync_copy(cacc.at[pl.ds(224, 40)], bounce.at[pl.ds(0, 40)])
        pltpu.sync_copy(bounce.at[pl.ds(0, 40)],
                        cnto_hbm.at[pl.ds(c * CAP + 224, 40)])


@functools.partial(
    pl.kernel,
    out_type=jax.ShapeDtypeStruct((NPAD, D), _f32),
    mesh=_mesh,
    compiler_params=_sc_params,
    scratch_types=(
        pltpu.VMEM((PT_N // 112, 112), _i32),
        pltpu.VMEM((2, 112, D), _f32),
        pltpu.SemaphoreType.DMA,
        pltpu.SemaphoreType.DMA,
    ),
)
def _gather(t1_hbm, x_hbm, out_hbm, xall, rows, sem, wsem):
    c = lax.axis_index("c")
    s = lax.axis_index("s")
    wid = s * NC + c
    NJ = PT_N // 112
    pltpu.sync_copy(x_hbm.at[wid], xall)
    pltpu.async_copy(t1_hbm.at[xall.at[0]], rows.at[0], sem)

    def body(t, carry):
        for p in (0, 1):
            jj = 2 * t + p
            pltpu.make_async_copy(t1_hbm.at[xall.at[0]], rows.at[p],
                                  sem).wait()

            @pl.when(jj >= 1)
            def _():
                pltpu.make_async_copy(rows.at[1 - p],
                                      out_hbm.at[pl.ds(0, 112)], wsem).wait()

            @pl.when(jj + 1 < NJ)
            def _():
                pltpu.async_copy(t1_hbm.at[xall.at[jj + 1]], rows.at[1 - p],
                                 sem)

            pltpu.async_copy(rows.at[p],
                             out_hbm.at[pl.ds(wid * PT_N + jj * 112, 112)],
                             wsem)
        return carry
    lax.fori_loop(0, NJ // 2, body, 0)
    pltpu.make_async_copy(rows.at[1], out_hbm.at[pl.ds(0, 112)], wsem).wait()


@functools.partial(
    pl.kernel,
    out_type=jax.ShapeDtypeStruct((2 * NPAD, DH), _f32),
    mesh=_mesh,
    compiler_params=_sc_params,
    scratch_types=(
        pltpu.VMEM((2, SLAB, EC), _i32),
        pltpu.VMEM((2, SLAB, EC), _i32),
        pltpu.VMEM((2 * GK * EC, DH), _f32),
        pltpu.VMEM_SHARED((NPAD, DH), _f32),
        pltpu.SemaphoreType.DMA,
        pltpu.SemaphoreType.DMA,
        pltpu.SemaphoreType.DMA,
    ),
)
def _agg(g_hbm, srcoff_hbm, dst_hbm, out_hbm,
         sidx, didx, rows, acc, sem, ssem, isem):
    c = lax.axis_index("c")
    s = lax.axis_index("s")
    _fill_rows(rows, 112, DH, 0.0)
    zrow = rows.at[pl.ds(0, 112)]

    def zacc(i, carry):
        pltpu.async_copy(zrow, acc.at[pl.ds(s * RPT + i * 112, 112)], isem)
        return carry
    lax.fori_loop(0, RPT // 112, zacc, 0)

    def zdrain(i, carry):
        pltpu.make_async_copy(zrow, acc.at[pl.ds(0, 112)], isem).wait()
        return carry
    lax.fori_loop(0, RPT // 112, zdrain, 0)

    plsc.subcore_barrier()

    def buf(k):
        return rows.at[pl.ds(k * EC, EC)]

    def fire(grp, par, q):
        for b in range(GK):
            pltpu.async_copy(g_hbm.at[sidx.at[q].at[grp * GK + b]],
                             buf(par * GK + b), sem)

    NGRP = SLAB // GK

    def drain_scatter(par):
        for b in range(GK):
            pltpu.make_async_copy(buf(par * GK + b),
                                  acc.at[pl.ds(0, EC)], ssem).wait()

    def slab_load(sl, q):
        pltpu.async_copy(srcoff_hbm.at[c, s, sl], sidx.at[q], isem)
        pltpu.async_copy(dst_hbm.at[s, sl], didx.at[q], isem)

    def slab_wait():
        pltpu.make_async_copy(srcoff_hbm.at[c, s, 0], sidx.at[0], isem).wait()
        pltpu.make_async_copy(dst_hbm.at[s, 0], didx.at[0], isem).wait()

    def do_slab(q, nxt):
        slab_wait()
        if nxt is not None:
            slab_load(nxt, 1 - q)
        fire(0, 0, q)

        def dbl(t, carry):
            for par in (0, 1):
                grp = t * 2 + par
                for b in range(GK):
                    pltpu.make_async_copy(g_hbm.at[pl.ds(0, EC)],
                                          buf(par * GK + b), sem).wait()

                @pl.when(grp >= 1)
                def _():
                    drain_scatter(1 - par)

                @pl.when(grp + 1 < NGRP)
                def _():
                    fire(grp + 1, 1 - par, q)

                for b in range(GK):
                    pltpu.async_copy(buf(par * GK + b),
                                     acc.at[didx.at[q].at[grp * GK + b]], ssem,
                                     add=True)
            return carry
        lax.fori_loop(0, NGRP // 2, dbl, 0)
        drain_scatter(1)

    slab_load(0, 0)

    def dslab(t, carry):
        do_slab(0, 2 * t + 1)
        do_slab(1, 2 * t + 2)
        return carry
    lax.fori_loop(0, (NSLAB - 1) // 2, dslab, 0)
    do_slab(0, None)

    plsc.subcore_barrier()

    NWO = RPT // 112
    wb = [rows.at[pl.ds(0, 112)], rows.at[pl.ds(112, 112)]]
    pltpu.async_copy(acc.at[pl.ds(s * RPT, 112)], wb[0], sem)

    def wout(t, carry):
        for p in (0, 1):
            i2 = 2 * t + p
            pltpu.make_async_copy(acc.at[pl.ds(0, 112)], wb[p], sem).wait()

            @pl.when(i2 >= 1)
            def _():
                pltpu.make_async_copy(wb[1 - p], out_hbm.at[pl.ds(0, 112)],
                                      ssem).wait()

            @pl.when(i2 + 1 < NWO)
            def _():
                pltpu.async_copy(
                    acc.at[pl.ds(s * RPT + (i2 + 1) * 112, 112)],
                    wb[1 - p], sem)

            pltpu.async_copy(wb[p],
                             out_hbm.at[pl.ds(c * NPAD + s * RPT + i2 * 112,
                                              112)], ssem)
        return carry
    lax.fori_loop(0, NWO // 2, wout, 0)
    pltpu.make_async_copy(wb[1], out_hbm.at[pl.ds(0, 112)], ssem).wait()


@functools.partial(
    pl.kernel,
    out_type=jax.ShapeDtypeStruct((2 * CAP, D), _f32),
    mesh=_mesh,
    compiler_params=_sc_params,
    scratch_types=(
        pltpu.VMEM((PT_N // 112, 112), _i32),
        pltpu.VMEM((2, 112, D), _f32),
        pltpu.VMEM((112, D), _f32),
        pltpu.VMEM_SHARED((CAP, D), _f32),
        pltpu.SemaphoreType.DMA,
        pltpu.SemaphoreType.DMA,
    ),
)
def _pool(h2_hbm, batch_hbm, out_hbm, ball, rows, zrow, pacc, sem, ssem):
    c = lax.axis_index("c")
    s = lax.axis_index("s")
    NJ = PT_N // 112
    base0 = c * (NPAD // 2) + s * PT_N
    _fill_rows(zrow, 112, D, 0.0)
    pltpu.sync_copy(batch_hbm.at[c, s], ball)

    @pl.when(s == 0)
    def _():
        pltpu.sync_copy(zrow, pacc.at[pl.ds(0, 112)])
        pltpu.sync_copy(zrow, pacc.at[pl.ds(112, 112)])
        pltpu.sync_copy(zrow.at[pl.ds(0, 40)], pacc.at[pl.ds(224, 40)])

    plsc.subcore_barrier()

    pltpu.async_copy(h2_hbm.at[pl.ds(base0, 112)], rows.at[0], sem)

    def body(t, carry):
        for p in (0, 1):
            jj = 2 * t + p
            pltpu.make_async_copy(h2_hbm.at[pl.ds(0, 112)], rows.at[p],
                                  sem).wait()

            @pl.when(jj >= 1)
            def _():
                pltpu.make_async_copy(rows.at[1 - p], pacc.at[pl.ds(0, 112)],
                                      ssem).wait()

            @pl.when(jj + 1 < NJ)
            def _():
                pltpu.async_copy(
                    h2_hbm.at[pl.ds(base0 + (jj + 1) * 112, 112)],
                    rows.at[1 - p], sem)

            pltpu.async_copy(rows.at[p], pacc.at[ball.at[jj]], ssem, add=True)
        return carry
    lax.fori_loop(0, NJ // 2, body, 0)
    pltpu.make_async_copy(rows.at[1], pacc.at[pl.ds(0, 112)], ssem).wait()

    plsc.subcore_barrier()

    @pl.when(s == 0)
    def _():
        pltpu.sync_copy(pacc.at[pl.ds(0, 112)], zrow)
        pltpu.sync_copy(zrow, out_hbm.at[pl.ds(c * CAP, 112)])
        pltpu.sync_copy(pacc.at[pl.ds(112, 112)], zrow)
        pltpu.sync_copy(zrow, out_hbm.at[pl.ds(c * CAP + 112, 112)])
        pltpu.sync_copy(pacc.at[pl.ds(224, 40)], zrow.at[pl.ds(0, 40)])
        pltpu.sync_copy(zrow.at[pl.ds(0, 40)],
                        out_hbm.at[pl.ds(c * CAP + 224, 40)])


RB = NPAD // 14


def _t1_body(emb_ref, w_ref, out_ref):
    out_ref[...] = lax.dot_general(emb_ref[...], w_ref[...],
                                   (((1,), (1,)), ((), ())),
                                   preferred_element_type=_f32)


def _t1_call(embed, W1):
    return pl.pallas_call(
        _t1_body,
        out_shape=jax.ShapeDtypeStruct((VOC, D), _f32),
    )(embed, W1)


def _t2_body(h_ref, d_ref, out_ref):
    dis = lax.rsqrt(d_ref[0] + d_ref[1] + 1.0)
    g = h_ref[...] * dis[:, None]
    out_ref[0] = g[:, :DH]
    out_ref[1] = g[:, DH:]


def _t2_call(hhat1, dpart):
    return pl.pallas_call(
        _t2_body,
        grid=(14,),
        in_specs=[
            pl.BlockSpec((RB, D), lambda i: (i, 0)),
            pl.BlockSpec((2, RB), lambda i: (0, i)),
        ],
        out_specs=pl.BlockSpec((2, RB, DH), lambda i: (0, i, 0)),
        out_shape=jax.ShapeDtypeStruct((2, NPAD, DH), _f32),
    )(hhat1, dpart)


def _t4_body(a_ref, g_ref, d_ref, w_ref, b_ref, out_ref):
    dis = lax.rsqrt(d_ref[0] + d_ref[1] + 1.0)
    agg = jnp.concatenate([a_ref[0], a_ref[1]], axis=1)
    g1 = jnp.concatenate([g_ref[0], g_ref[1]], axis=1)
    h1 = jnp.maximum((agg + g1) * dis[:, None] + b_ref[...], 0.0)
    hh2 = lax.dot_general(h1, w_ref[...], (((1,), (1,)), ((), ())),
                          preferred_element_type=_f32)
    g2 = hh2 * dis[:, None]
    out_ref[0] = g2[:, :DH]
    out_ref[1] = g2[:, DH:]


def _t4_call(agg1, g1, dpart, W2, b1):
    return pl.pallas_call(
        _t4_body,
        grid=(14,),
        in_specs=[
            pl.BlockSpec((2, RB, DH), lambda i: (0, i, 0)),
            pl.BlockSpec((2, RB, DH), lambda i: (0, i, 0)),
            pl.BlockSpec((2, RB), lambda i: (0, i)),
            pl.BlockSpec((D, D), lambda i: (0, 0)),
            pl.BlockSpec((1, D), lambda i: (0, 0)),
        ],
        out_specs=pl.BlockSpec((2, RB, DH), lambda i: (0, i, 0)),
        out_shape=jax.ShapeDtypeStruct((2, NPAD, DH), _f32),
    )(agg1, g1, dpart, W2, b1)


def _t5_body(a_ref, g_ref, d_ref, b_ref, out_ref):
    dis = lax.rsqrt(d_ref[0] + d_ref[1] + 1.0)
    agg = jnp.concatenate([a_ref[0], a_ref[1]], axis=1)
    g2 = jnp.concatenate([g_ref[0], g_ref[1]], axis=1)
    out_ref[...] = jnp.maximum((agg + g2) * dis[:, None] + b_ref[...], 0.0)


def _t5_call(agg2, g2, dpart, b2):
    return pl.pallas_call(
        _t5_body,
        grid=(14,),
        in_specs=[
            pl.BlockSpec((2, RB, DH), lambda i: (0, i, 0)),
            pl.BlockSpec((2, RB, DH), lambda i: (0, i, 0)),
            pl.BlockSpec((2, RB), lambda i: (0, i)),
            pl.BlockSpec((1, D), lambda i: (0, 0)),
        ],
        out_specs=pl.BlockSpec((RB, D), lambda i: (i, 0)),
        out_shape=jax.ShapeDtypeStruct((NPAD, D), _f32),
    )(agg2, g2, dpart, b2)


def _t6_body(p_ref, c_ref, w_ref, b_ref, out_ref):
    sums = p_ref[0, :G, :] + p_ref[1, :G, :]
    cnt = c_ref[0, :G] + c_ref[1, :G]
    pooled = sums / jnp.maximum(cnt, 1.0)[:, None]
    out_ref[...] = lax.dot_general(pooled, w_ref[...],
                                   (((1,), (1,)), ((), ())),
                                   preferred_element_type=_f32) + b_ref[...]


def _t6_call(ppart, cpart, linW, linb):
    return pl.pallas_call(
        _t6_body,
        out_shape=jax.ShapeDtypeStruct((G, NCLS), _f32),
    )(ppart, cpart, linW, linb)


@jax.jit
def kernel(x, edge_index, batch, embed, W1, b1, W2, b2, linW, linb):
    x = x.astype(_i32)
    src = edge_index[0].astype(_i32)
    dst = edge_index[1].astype(_i32)
    batch = batch.astype(_i32)

    x_pad = jnp.concatenate([x, jnp.zeros((NPAD - N,), _i32)])
    src_pad = jnp.concatenate([src, jnp.zeros((EPAD - E,), _i32)])
    dst_pad = jnp.concatenate([dst, jnp.full((EPAD - E,), N, _i32)])
    srcoff = jnp.concatenate([src_pad, src_pad + NPAD])
    batch_pad = jnp.concatenate([batch, jnp.full((NPAD - N,), G, _i32)])

    dego, cnto = _counts(dst_pad.reshape(2, NS, NCH0, EC),
                         batch_pad.reshape(2, NS, PT_N // 112, 112))
    dpart = dego.reshape(2, NPAD)

    t1 = _t1_call(embed, W1)
    hhat1 = _gather(t1, x_pad.reshape(NW, PT_N // 112, 112))
    g1 = _t2_call(hhat1, dpart)

    srcoff4 = srcoff.reshape(2, NS, NSLAB, SLAB, EC)
    dst4 = dst_pad.reshape(NS, NSLAB, SLAB, EC)
    agg1 = _agg(g1.reshape(2 * NPAD, DH), srcoff4, dst4).reshape(2, NPAD, DH)
    g2 = _t4_call(agg1, g1, dpart, W2, b1.reshape(1, D))

    agg2 = _agg(g2.reshape(2 * NPAD, DH), srcoff4, dst4).reshape(2, NPAD, DH)
    h2 = _t5_call(agg2, g2, dpart, b2.reshape(1, D))

    ppart = _pool(h2, batch_pad.reshape(2, NS, PT_N // 112, 112))
    return _t6_call(ppart.reshape(2, CAP, D), cnto.reshape(2, CAP),
                    linW, linb.reshape(1, NCLS))
